# Initial kernel scaffold; baseline (speedup 1.0000x reference)
import jax
import jax.numpy as jnp
from jax.experimental import pallas as pl


def kernel(x, edge_index, edge_attr, batch, emb, W1_0, b1_0, W2_0, b2_0, W1_1, b1_1, W2_1, b2_1, W1_2, b1_2, W2_2, b2_2, Wp, bp):
    raise NotImplementedError("write your pallas kernel here")


# speedup vs baseline: 4.9483x; 4.9483x over previous
import functools

import jax
import jax.numpy as jnp
from jax import lax
from jax.experimental import pallas as pl
from jax.experimental.pallas import tpu as pltpu
from jax.experimental.pallas import tpu_sc as plsc

_N = 10000
_E = 320000
_H = 128
_G = 128
_C = 10

_NC = 2
_NS = 16
_NW = _NC * _NS
_NPAD = 10240
_ROWS_PER_TILE = _NPAD // _NS
_EDGES_PER_SC = _E // _NC
_EDGES_PER_TILE = _E // _NW
_K = 80
_CHUNKS = _EDGES_PER_TILE // _K


def _segment_sum_sc(h, src, dst, zeros):
    mesh = plsc.VectorSubcoreMesh(core_axis_name="c", subcore_axis_name="s")

    @functools.partial(
        pl.kernel,
        out_type=jax.ShapeDtypeStruct((_NC, _NPAD, _H), jnp.float32),
        mesh=mesh,
        scratch_types=[
            pltpu.VMEM((_K,), jnp.int32),
            pltpu.VMEM((_K,), jnp.int32),
            pltpu.VMEM((_K, _H), jnp.float32),
            pltpu.VMEM_SHARED((_NPAD, _H), jnp.float32),
            pltpu.SemaphoreType.DMA,
        ],
    )
    def seg_kernel(h_hbm, src_hbm, dst_hbm, z_hbm, out_hbm,
                   idx_s, idx_d, msgs, agg, sem):
        c = lax.axis_index("c")
        s = lax.axis_index("s")
        row0 = s * _ROWS_PER_TILE
        pltpu.sync_copy(z_hbm, agg.at[pl.ds(row0, _ROWS_PER_TILE)])
        plsc.subcore_barrier()

        ebase = c * _EDGES_PER_SC + s * _EDGES_PER_TILE

        def body(i, carry):
            off = ebase + i * _K
            pltpu.sync_copy(src_hbm.at[pl.ds(off, _K)], idx_s)
            pltpu.sync_copy(dst_hbm.at[pl.ds(off, _K)], idx_d)
            pltpu.async_copy(h_hbm.at[idx_s], msgs, sem).wait()
            pltpu.sync_copy(msgs, agg.at[idx_d], add=True)
            return carry

        lax.fori_loop(0, _CHUNKS, body, 0)
        plsc.subcore_barrier()
        pltpu.sync_copy(agg.at[pl.ds(row0, _ROWS_PER_TILE)],
                        out_hbm.at[c, pl.ds(row0, _ROWS_PER_TILE)])

    return seg_kernel(h, src, dst, zeros)


_BLK = 1000
_NBLK = _N // _BLK


def _mlp_body(h_ref, agg_ref, w1_ref, b1_ref, w2_ref, b2_ref, out_ref):
    z = h_ref[...] + agg_ref[0] + agg_ref[1]
    t = jnp.maximum(
        jnp.dot(z, w1_ref[...], preferred_element_type=jnp.float32)
        + b1_ref[...], 0.0)
    out_ref[...] = (
        jnp.dot(t, w2_ref[...], preferred_element_type=jnp.float32)
        + b2_ref[...])


def _gin_mlp_tc(h, agg_parts, w1, b1, w2, b2):
    return pl.pallas_call(
        _mlp_body,
        grid=(_NBLK,),
        in_specs=[
            pl.BlockSpec((_BLK, _H), lambda i: (i, 0)),
            pl.BlockSpec((_NC, _BLK, _H), lambda i: (0, i, 0)),
            pl.BlockSpec((_H, _H), lambda i: (0, 0)),
            pl.BlockSpec((1, _H), lambda i: (0, 0)),
            pl.BlockSpec((_H, _H), lambda i: (0, 0)),
            pl.BlockSpec((1, _H), lambda i: (0, 0)),
        ],
        out_specs=pl.BlockSpec((_BLK, _H), lambda i: (i, 0)),
        out_shape=jax.ShapeDtypeStruct((_N, _H), jnp.float32),
    )(h, agg_parts, w1, b1.reshape(1, _H), w2, b2.reshape(1, _H))


def _pool_body(batch_ref, h_ref, wp_ref, bp_ref, out_ref, sum_ref, cnt_ref):
    i = pl.program_id(0)

    @pl.when(i == 0)
    def _init():
        sum_ref[...] = jnp.zeros_like(sum_ref)
        cnt_ref[...] = jnp.zeros_like(cnt_ref)

    b = jnp.broadcast_to(batch_ref[...].reshape(1, _BLK), (_G, _BLK))
    gid = lax.broadcasted_iota(jnp.int32, (_G, _BLK), 0)
    onehot_t = (b == gid).astype(jnp.float32)
    sum_ref[...] += jnp.dot(onehot_t, h_ref[...],
                            preferred_element_type=jnp.float32)
    cnt_ref[...] += jnp.sum(onehot_t, axis=1, keepdims=True)

    @pl.when(i == _NBLK - 1)
    def _fin():
        pooled = sum_ref[...] / jnp.maximum(cnt_ref[...], 1.0)
        out_ref[...] = (
            jnp.dot(pooled, wp_ref[...], preferred_element_type=jnp.float32)
            + bp_ref[...])


def _pool_head_tc(h3, batch3d, wp, bp):
    return pl.pallas_call(
        _pool_body,
        grid=(_NBLK,),
        in_specs=[
            pl.BlockSpec((1, 1, _BLK), lambda i: (i, 0, 0)),
            pl.BlockSpec((_BLK, _H), lambda i: (i, 0)),
            pl.BlockSpec((_H, _C), lambda i: (0, 0)),
            pl.BlockSpec((1, _C), lambda i: (0, 0)),
        ],
        out_specs=pl.BlockSpec((_G, _C), lambda i: (0, 0)),
        out_shape=jax.ShapeDtypeStruct((_G, _C), jnp.float32),
        scratch_shapes=[
            pltpu.VMEM((_G, _H), jnp.float32),
            pltpu.VMEM((_G, 1), jnp.float32),
        ],
    )(batch3d, h3, wp, bp.reshape(1, _C))


def kernel(x, edge_index, edge_attr, batch, emb,
           W1_0, b1_0, W2_0, b2_0,
           W1_1, b1_1, W2_1, b2_1,
           W1_2, b1_2, W2_2, b2_2,
           Wp, bp):
    src = edge_index[0].astype(jnp.int32)
    dst = edge_index[1].astype(jnp.int32)
    zeros = jnp.zeros((_ROWS_PER_TILE, _H), jnp.float32)
    h = jnp.broadcast_to(emb[0].astype(jnp.float32), (_N, _H))
    batch3d = batch.astype(jnp.int32).reshape(_NBLK, 1, _BLK)

    for (w1, b1, w2, b2) in ((W1_0, b1_0, W2_0, b2_0),
                             (W1_1, b1_1, W2_1, b2_1),
                             (W1_2, b1_2, W2_2, b2_2)):
        agg_parts = _segment_sum_sc(h, src, dst, zeros)[:, :_N]
        h = _gin_mlp_tc(h, agg_parts, w1, b1, w2, b2)

    return _pool_head_tc(h, batch3d, Wp, bp)
